# Initial kernel scaffold; baseline (speedup 1.0000x reference)
import jax
import jax.numpy as jnp
from jax.experimental import pallas as pl


def kernel(x, W1, b1, W2, b2, Wout, bout, a1, bh1, a2, bh2):
    raise NotImplementedError("write your pallas kernel here")


# speedup vs baseline: 2.2652x; 2.2652x over previous
import jax
import jax.numpy as jnp
from jax.experimental import pallas as pl
from jax.experimental.pallas import tpu as pltpu

_R = 0.1
_EPS = 1e-12
_M = 5

_INTERPRET = False


def _hash_int(val):
    h = jnp.floor(val / _R).astype(jnp.int32)
    return jnp.bitwise_and(h, 1)


def _kernel_hash(wm, av, bh, kdim):
    sq = jnp.sum(wm * wm, axis=1, keepdims=True)
    n = jnp.sqrt(sq)
    maxn = jnp.max(n, axis=0, keepdims=True) + _EPS
    s = 0.75 / maxn
    dots = jnp.sum(wm * av[:, :kdim], axis=1, keepdims=True)
    val = s * dots + bh
    p = (s * n) * (s * n)
    for i in range(_M):
        val = val + p * av[:, kdim + i:kdim + i + 1]
        p = p * p
    return _hash_int(val)


def _query_hash(qms, av, bh, cin):
    rn = qms[0] * qms[0]
    for c in range(1, cin):
        rn = rn + qms[c] * qms[c]
    denom = 3.0 * jnp.sqrt(rn) + _EPS
    qdot = jnp.zeros_like(qms[0])
    for c in range(cin):
        ga = jnp.sum(av[:, 9 * c:9 * c + 9], axis=1, keepdims=True)
        qdot = qdot + qms[c] * ga
    kdim = 9 * cin
    halves = 0.5 * jnp.sum(av[:, kdim:kdim + _M], axis=1, keepdims=True)
    return _hash_int(qdot / denom + halves + bh)


def _mask1_kernel(x_ref, w1m_ref, a1_ref, bh1_ref, m1_ref):
    wm = w1m_ref[...]
    av = a1_ref[...]
    bh = bh1_ref[...]
    kh = _kernel_hash(wm, av, bh, 27)
    qms = []
    for c in range(3):
        t = jnp.sum(x_ref[0, c], axis=1, keepdims=True)
        qms.append(jnp.sum(t, axis=0, keepdims=True) / (224.0 * 1004.0))
    qh = _query_hash(qms, av, bh, 3)
    row = jnp.concatenate(
        [(kh[o:o + 1, :] == qh).astype(jnp.int32) for o in range(5)], axis=1)
    m1_ref[0] = row


def _conv1_kernel(x_ref, w1s_ref, b1s_ref, m1s_ref, h1_ref, qs2_ref):
    b = pl.program_id(0)
    xs = [[x_ref[0, c, :, dx:dx + 1002] for dx in range(3)] for c in range(3)]
    qs2_ref[0] = jnp.zeros((1, 5), jnp.float32)
    for o in range(5):
        @pl.when(m1s_ref[b, o] != 0)
        def _(o=o):
            acc = jnp.full((222, 1002), b1s_ref[o], jnp.float32)
            for c in range(3):
                for dy in range(3):
                    for dx in range(3):
                        w = w1s_ref[o * 27 + c * 9 + dy * 3 + dx]
                        acc = acc + w * xs[c][dx][dy:dy + 222, :]
            hv = jnp.maximum(acc, 0.0)
            h1_ref[0, o] = hv
            t = jnp.sum(hv, axis=1, keepdims=True)
            qs2_ref[0, 0:1, o:o + 1] = jnp.sum(t, axis=0, keepdims=True)

        @pl.when(m1s_ref[b, o] == 0)
        def _(o=o):
            h1_ref[0, o] = jnp.zeros((222, 1002), jnp.float32)


def _mask2_kernel(qs2_ref, w2m_ref, a2_ref, bh2_ref, m2_ref):
    wm = w2m_ref[...]
    av = a2_ref[...]
    bh = bh2_ref[...]
    kh = _kernel_hash(wm, av, bh, 45)
    qm = qs2_ref[...] / (222.0 * 1002.0)
    qms = [qm[:, c:c + 1] for c in range(5)]
    qh = _query_hash(qms, av, bh, 5)
    m2_ref[...] = jnp.concatenate(
        [(qh == kh[o:o + 1, :]).astype(jnp.int32) for o in range(5)], axis=1)


def _conv2_kernel(h1_ref, w2s_ref, b2s_ref, m1s_ref, m2s_ref, wout_ref,
                  bout_ref, out_ref, acc_ref):
    b = pl.program_id(0)
    for o in range(5):
        @pl.when(m2s_ref[b, o] != 0)
        def _(o=o):
            acc_ref[o] = jnp.full((220, 1000), b2s_ref[o], jnp.float32)
    for c in range(5):
        @pl.when(m1s_ref[b, c] != 0)
        def _(c=c):
            hs = [h1_ref[0, c, :, dx:dx + 1000] for dx in range(3)]
            for o in range(5):
                @pl.when(m2s_ref[b, o] != 0)
                def _(o=o, c=c):
                    v = w2s_ref[o * 45 + c * 9] * hs[0][0:220, :]
                    for dy in range(3):
                        for dx in range(3):
                            if dy == 0 and dx == 0:
                                continue
                            w = w2s_ref[o * 45 + c * 9 + dy * 3 + dx]
                            v = v + w * hs[dx][dy:dy + 220, :]
                    acc_ref[o] = acc_ref[o] + v
    wout = wout_ref[...]
    bout = bout_ref[...]
    for o in range(5):
        @pl.when(m2s_ref[b, o] != 0)
        def _(o=o):
            h2 = jnp.maximum(acc_ref[o], 0.0)
            out_ref[0, o] = jax.lax.dot_general(
                h2, wout, (((1,), (0,)), ((), ())),
                preferred_element_type=jnp.float32) + bout

        @pl.when(m2s_ref[b, o] == 0)
        def _(o=o):
            out_ref[0, o] = jnp.broadcast_to(bout, (220, 10))


def kernel(x, W1, b1, W2, b2, Wout, bout, a1, bh1, a2, bh2):
    B, C, H, W = x.shape
    w1m = W1.reshape(5, 27)
    w2m = W2.reshape(5, 45)
    a1v = a1.reshape(1, 32)
    a2v = a2.reshape(1, 50)
    bh1v = bh1.reshape(1, 1)
    bh2v = bh2.reshape(1, 1)

    m1 = pl.pallas_call(
        _mask1_kernel,
        grid=(B,),
        in_specs=[
            pl.BlockSpec((1, C, H, W), lambda b: (b, 0, 0, 0)),
            pl.BlockSpec((5, 27), lambda b: (0, 0)),
            pl.BlockSpec((1, 32), lambda b: (0, 0)),
            pl.BlockSpec((1, 1), lambda b: (0, 0)),
        ],
        out_specs=pl.BlockSpec((1, 1, 5), lambda b: (b, 0, 0)),
        out_shape=jax.ShapeDtypeStruct((B, 1, 5), jnp.int32),
        interpret=_INTERPRET,
    )(x, w1m, a1v, bh1v)
    m1 = m1.reshape(B, 5)

    h1, qs2 = pl.pallas_call(
        _conv1_kernel,
        grid=(B,),
        in_specs=[
            pl.BlockSpec((1, C, H, W), lambda b: (b, 0, 0, 0)),
            pl.BlockSpec(memory_space=pltpu.SMEM),
            pl.BlockSpec(memory_space=pltpu.SMEM),
            pl.BlockSpec(memory_space=pltpu.SMEM),
        ],
        out_specs=[
            pl.BlockSpec((1, 5, 222, 1002), lambda b: (b, 0, 0, 0)),
            pl.BlockSpec((1, 1, 5), lambda b: (b, 0, 0)),
        ],
        out_shape=[
            jax.ShapeDtypeStruct((B, 5, 222, 1002), jnp.float32),
            jax.ShapeDtypeStruct((B, 1, 5), jnp.float32),
        ],
        interpret=_INTERPRET,
    )(x, W1.reshape(135), b1, m1)
    qs2 = qs2.reshape(B, 5)

    m2 = pl.pallas_call(
        _mask2_kernel,
        in_specs=[
            pl.BlockSpec((16, 5), lambda: (0, 0)),
            pl.BlockSpec((5, 45), lambda: (0, 0)),
            pl.BlockSpec((1, 50), lambda: (0, 0)),
            pl.BlockSpec((1, 1), lambda: (0, 0)),
        ],
        out_specs=pl.BlockSpec((16, 5), lambda: (0, 0)),
        out_shape=jax.ShapeDtypeStruct((B, 5), jnp.int32),
        interpret=_INTERPRET,
    )(qs2, w2m, a2v, bh2v)

    out = pl.pallas_call(
        _conv2_kernel,
        grid=(B,),
        in_specs=[
            pl.BlockSpec((1, 5, 222, 1002), lambda b: (b, 0, 0, 0)),
            pl.BlockSpec(memory_space=pltpu.SMEM),
            pl.BlockSpec(memory_space=pltpu.SMEM),
            pl.BlockSpec(memory_space=pltpu.SMEM),
            pl.BlockSpec(memory_space=pltpu.SMEM),
            pl.BlockSpec((1000, 10), lambda b: (0, 0)),
            pl.BlockSpec((1, 10), lambda b: (0, 0)),
        ],
        out_specs=pl.BlockSpec((1, 5, 220, 10), lambda b: (b, 0, 0, 0)),
        out_shape=jax.ShapeDtypeStruct((B, 5, 220, 10), jnp.float32),
        scratch_shapes=[pltpu.VMEM((5, 220, 1000), jnp.float32)],
        interpret=_INTERPRET,
    )(h1, W2.reshape(225), b2, m1, m2, Wout, bout.reshape(1, 10))

    return out


# speedup vs baseline: 7.6786x; 3.3898x over previous
import jax
import jax.numpy as jnp
from jax.experimental import pallas as pl
from jax.experimental.pallas import tpu as pltpu

_R = 0.1
_EPS = 1e-12
_M = 5

_INTERPRET = False


def _tsum(terms):
    while len(terms) > 1:
        nxt = [terms[i] + terms[i + 1] for i in range(0, len(terms) - 1, 2)]
        if len(terms) % 2:
            nxt.append(terms[-1])
        terms = nxt
    return terms[0]


def _hash_scalar(val):
    return jnp.bitwise_and(jnp.floor(val / _R).astype(jnp.int32), 1)


def _kernel_hashes(wf_ref, af_ref, bhs, nout, kdim):
    sqs, dots = [], []
    for o in range(nout):
        base = o * kdim
        sq = wf_ref[base] * wf_ref[base]
        dot = wf_ref[base] * af_ref[0]
        for k in range(1, kdim):
            w = wf_ref[base + k]
            sq = sq + w * w
            dot = dot + w * af_ref[k]
        sqs.append(sq)
        dots.append(dot)
    maxsq = sqs[0]
    for o in range(1, nout):
        maxsq = jnp.maximum(maxsq, sqs[o])
    maxn = jnp.sqrt(maxsq) + _EPS
    s = 0.75 / maxn
    khs = []
    for o in range(nout):
        val = s * dots[o] + bhs
        p = (s * s) * sqs[o]
        for i in range(_M):
            val = val + p * af_ref[kdim + i]
            p = p * p
        khs.append(_hash_scalar(val))
    return khs


def _query_hash(qms, af_ref, bhs, cin):
    rn = qms[0] * qms[0]
    for c in range(1, cin):
        rn = rn + qms[c] * qms[c]
    denom = 3.0 * jnp.sqrt(rn) + _EPS
    qdot = qms[0] * 0.0
    for c in range(cin):
        ga = af_ref[9 * c]
        for k in range(1, 9):
            ga = ga + af_ref[9 * c + k]
        qdot = qdot + qms[c] * ga
    kdim = 9 * cin
    halves = af_ref[kdim]
    for i in range(1, _M):
        halves = halves + af_ref[kdim + i]
    return _hash_scalar(qdot / denom + 0.5 * halves + bhs)


def _fused_kernel(x_ref, w1f_ref, b1f_ref, w2f_ref, b2f_ref, a1_ref, bh1_ref,
                  a2_ref, bh2_ref, wout_ref, bout_ref, out_ref,
                  xsh_ref, h1_ref, h1sh_ref, acc2_ref, qs_ref):
    kh1 = _kernel_hashes(w1f_ref, a1_ref, bh1_ref[0], 5, 27)
    qm1 = [jnp.sum(x_ref[0, c]) * (1.0 / (224.0 * 1004.0)) for c in range(3)]
    qh1 = _query_hash(qm1, a1_ref, bh1_ref[0], 3)
    m1 = [kh1[o] == qh1 for o in range(5)]

    for c in range(3):
        for dx in (1, 2):
            xsh_ref[c, dx - 1] = x_ref[0, c, :, dx:dx + 1002]

    def x_tile(c, dx, t):
        if dx == 0:
            return x_ref[0, c, 8 * t:8 * t + 8, 0:1002]
        return xsh_ref[c, dx - 1, 8 * t:8 * t + 8, :]

    for o in range(5):
        @pl.when(m1[o])
        def _(o=o):
            ws = [[[w1f_ref[o * 27 + c * 9 + dy * 3 + dx]
                    for dx in range(3)] for dy in range(3)] for c in range(3)]

            def compute_s(t):
                xt = [[x_tile(c, dx, t) for dx in range(3)] for c in range(3)]
                return [_tsum([ws[c][dy][dx] * xt[c][dx]
                               for c in range(3) for dx in range(3)])
                        for dy in range(3)]

            sc = compute_s(0)
            tot = qh1.astype(jnp.float32) * 0.0
            for t in range(28):
                if t < 27:
                    sn = compute_s(t + 1)
                    v = _tsum([sc[0]] + [jnp.concatenate(
                        [sc[dy][dy:, :], sn[dy][:dy, :]], axis=0)
                        for dy in (1, 2)])
                    rows = 8
                else:
                    sn = None
                    v = _tsum([sc[0][0:6, :]] + [sc[dy][dy:dy + 6, :]
                                                 for dy in (1, 2)])
                    rows = 6
                v = jnp.maximum(v + b1f_ref[o], 0.0)
                h1_ref[o, 8 * t:8 * t + rows, :] = v
                for dx in (1, 2):
                    h1sh_ref[o, dx - 1, 8 * t:8 * t + rows, :] = \
                        v[:, dx:dx + 1000]
                tot = tot + jnp.sum(v)
                sc = sn
            qs_ref[o] = tot

        @pl.when(jnp.logical_not(m1[o]))
        def _(o=o):
            qs_ref[o] = 0.0

    kh2 = _kernel_hashes(w2f_ref, a2_ref, bh2_ref[0], 5, 45)
    qm2 = [qs_ref[c] * (1.0 / (222.0 * 1002.0)) for c in range(5)]
    qh2 = _query_hash(qm2, a2_ref, bh2_ref[0], 5)
    m2 = [kh2[o] == qh2 for o in range(5)]

    def h1_tile(c, dx, t):
        if dx == 0:
            return h1_ref[c, 8 * t:8 * t + 8, 0:1000]
        return h1sh_ref[c, dx - 1, 8 * t:8 * t + 8, :]

    bout = bout_ref[...]

    for o in range(5):
        @pl.when(m2[o])
        def _(o=o):
            acc2_ref[...] = jnp.full((220, 1000), b2f_ref[o], jnp.float32)
            for ci in range(5):
                @pl.when(m1[ci])
                def _(o=o, ci=ci):
                    ws = [[w2f_ref[o * 45 + ci * 9 + dy * 3 + dx]
                           for dx in range(3)] for dy in range(3)]

                    def compute_s(t):
                        ht = [h1_tile(ci, dx, t) for dx in range(3)]
                        return [_tsum([ws[dy][dx] * ht[dx]
                                       for dx in range(3)])
                                for dy in range(3)]

                    sc = compute_s(0)
                    for t in range(28):
                        if t < 27:
                            sn = compute_s(t + 1)
                            v = _tsum([sc[0]] + [jnp.concatenate(
                                [sc[dy][dy:, :], sn[dy][:dy, :]], axis=0)
                                for dy in (1, 2)])
                            rows = 8
                        else:
                            sn = None
                            v = _tsum([sc[0][0:4, :]] + [sc[dy][dy:dy + 4, :]
                                                         for dy in (1, 2)])
                            rows = 4
                        sl = pl.ds(8 * t, rows)
                        acc2_ref[sl, :] = acc2_ref[sl, :] + v
                        sc = sn

            h2 = jnp.maximum(acc2_ref[...], 0.0)
            out_ref[0, o] = jax.lax.dot_general(
                h2, wout_ref[...], (((1,), (0,)), ((), ())),
                preferred_element_type=jnp.float32) + bout

        @pl.when(jnp.logical_not(m2[o]))
        def _(o=o):
            out_ref[0, o] = jnp.broadcast_to(bout, (220, 10))


def kernel(x, W1, b1, W2, b2, Wout, bout, a1, bh1, a2, bh2):
    B, C, H, W = x.shape
    smem = pl.BlockSpec(memory_space=pltpu.SMEM)
    out = pl.pallas_call(
        _fused_kernel,
        grid=(B,),
        in_specs=[
            pl.BlockSpec((1, C, H, W), lambda b: (b, 0, 0, 0)),
            smem, smem, smem, smem, smem, smem, smem, smem,
            pl.BlockSpec((1000, 10), lambda b: (0, 0)),
            pl.BlockSpec((1, 10), lambda b: (0, 0)),
        ],
        out_specs=pl.BlockSpec((1, 5, 220, 10), lambda b: (b, 0, 0, 0)),
        out_shape=jax.ShapeDtypeStruct((B, 5, 220, 10), jnp.float32),
        scratch_shapes=[
            pltpu.VMEM((3, 2, 224, 1002), jnp.float32),
            pltpu.VMEM((5, 224, 1002), jnp.float32),
            pltpu.VMEM((5, 2, 224, 1000), jnp.float32),
            pltpu.VMEM((220, 1000), jnp.float32),
            pltpu.SMEM((5,), jnp.float32),
        ],
        interpret=_INTERPRET,
    )(x, W1.reshape(135), b1, W2.reshape(225), b2, a1, bh1.reshape(1),
      a2, bh2.reshape(1), Wout, bout.reshape(1, 10))
    return out
